# baseline (speedup 1.0000x reference)
import functools

import jax
import jax.numpy as jnp
import numpy as np
from jax.experimental import pallas as pl
from jax.experimental.pallas import tpu as pltpu

N_BINS = 10
A_PAD = 2048
R = 1024
TWO_PI = np.float32(2.0 * np.pi)

_I0E_COEFFS_A = (
    -1.30002500998624804212E-8, 6.04699502254191894932E-8,
    -2.67079385394061173391E-7, 1.11738753912010371815E-6,
    -4.41673835845875056359E-6, 1.64484480707288970893E-5,
    -5.75419501008210370398E-5, 1.88502885095841655729E-4,
    -5.76375574538582365885E-4, 1.63947561694133579842E-3,
    -4.32430999505057594430E-3, 1.05464603945949983183E-2,
    -2.37374148058994688156E-2, 4.93052842396707084878E-2,
    -9.49010970480476444210E-2, 1.71620901522208775349E-1,
    -3.04682672343198398683E-1, 6.76795274409476084995E-1)
_I0E_COEFFS_B = (
    3.39623202570838634515E-9, 2.26666899049817806459E-8,
    2.04891858946906374183E-7, 2.89137052083475648297E-6,
    6.88975834691682398426E-5, 3.36911647825569408990E-3,
    8.04490411014108831608E-1)


def _chbevl(x, coefficients):
    b0 = jnp.zeros_like(x)
    b1 = jnp.zeros_like(x)
    b2 = jnp.zeros_like(x)
    for c in coefficients:
        b2 = b1
        b1 = b0
        b0 = x * b1 - b2 + jnp.float32(c)
    return 0.5 * (b0 - b2)


def _i0e(x):
    x = jnp.abs(x)
    result_le_8 = _chbevl(0.5 * x - 2.0, _I0E_COEFFS_A)
    result_gt_8 = _chbevl(32.0 / x - 2.0, _I0E_COEFFS_B) / jnp.sqrt(x)
    return jnp.where(x <= 8.0, result_le_8, result_gt_8)


def _log_i0(x):
    return jnp.log(_i0e(x)) + jnp.abs(x)


def _log_cosh(x):
    ax = jnp.abs(x)
    return ax + jnp.log1p(jnp.exp(-2.0 * ax)) - jnp.log(2.0).astype(jnp.float32)


def _f2_kernel(hkl_ref, frac_ref, fvec_ref, f2_ref):
    dn = (((1,), (0,)), ((), ()))
    praw = jax.lax.dot_general(hkl_ref[:, :], frac_ref[:, :], dn,
                               preferred_element_type=jnp.float32)
    phase = TWO_PI * praw
    c = jnp.cos(phase)
    s = jnp.sin(phase)
    A = jax.lax.dot_general(c, fvec_ref[:, :], dn,
                            precision=jax.lax.Precision.HIGHEST,
                            preferred_element_type=jnp.float32)
    B = jax.lax.dot_general(s, fvec_ref[:, :], dn,
                            precision=jax.lax.Precision.HIGHEST,
                            preferred_element_type=jnp.float32)
    f2_ref[:, :] = A * A + B * B


def _stats_kernel(f2_ref, emean_ref, dobs_ref, cent_ref, bins_ref, out_ref):
    if True:
        F2 = f2_ref[:, :]
        bins = bins_ref[:, :]
        Em = emean_ref[:, :]
        Do = dobs_ref[:, :]
        ce = cent_ref[:, :]
        Eobs2 = Em * Em

        zero = jnp.zeros_like(F2)
        ns, muF2s, muOs = [], [], []
        for b in range(N_BINS):
            oh = bins == b
            n = jnp.sum(jnp.where(oh, 1.0, 0.0))
            sF2 = jnp.sum(jnp.where(oh, F2, zero))
            sO = jnp.sum(jnp.where(oh, Eobs2, zero))
            ns.append(n)
            muF2s.append(sF2 / n)
            muOs.append(sO / n)
        g_sqF2 = jnp.ones_like(F2)
        g_muO = jnp.zeros_like(F2)
        for b in range(N_BINS):
            oh = bins == b
            g_sqF2 = jnp.where(oh, jnp.sqrt(muF2s[b]), g_sqF2)
            g_muO = jnp.where(oh, muOs[b], g_muO)
        Ecalc = jnp.sqrt(F2 + 1e-12) / g_sqF2
        Ecalc2 = Ecalc * Ecalc
        Eoi = Eobs2 - g_muO

        muCs = []
        for b in range(N_BINS):
            oh = bins == b
            muCs.append(jnp.sum(jnp.where(oh, Ecalc2, zero)) / ns[b])
        g_muC = jnp.zeros_like(F2)
        for b in range(N_BINS):
            g_muC = jnp.where(bins == b, muCs[b], g_muC)
        Eci = Ecalc2 - g_muC

        g_sig = jnp.full_like(F2, 0.5)
        for b in range(N_BINS):
            oh = bins == b
            n = ns[b]
            cov = jnp.sum(jnp.where(oh, Eci * Eoi, zero)) / (n - 1.0)
            m_o = jnp.sum(jnp.where(oh, Eoi, zero)) / n
            m_c = jnp.sum(jnp.where(oh, Eci, zero)) / n
            sso = jnp.sum(jnp.where(oh, Eoi * Eoi, zero)) / n
            ssc = jnp.sum(jnp.where(oh, Eci * Eci, zero)) / n
            std_o = jnp.sqrt(sso - m_o * m_o)
            std_c = jnp.sqrt(ssc - m_c * m_c)
            sig = jnp.sqrt(jnp.clip(cov / (std_o * std_c), 0.001, 0.999))
            g_sig = jnp.where(oh, sig, g_sig)

        dsA = Do * g_sig
        t2 = dsA * dsA
        den = 1.0 - t2
        e2sum = Em ** 2 + Ecalc ** 2
        barg = 2.0 * Em * Ecalc * dsA / den
        llg_a = -jnp.log(den) - t2 * e2sum / den + _log_i0(barg)
        carg = Em * Ecalc * dsA / den
        llg_c = -0.5 * jnp.log(den) - t2 * e2sum / (2.0 * den) + _log_cosh(carg)
        llg = jnp.where(ce > 0.5, llg_c, llg_a)
        total = jnp.sum(jnp.where(bins < N_BINS, llg, zero))
        out_ref[0, 0] = total


def kernel(xyz_ort, f_atoms, Emean, Dobs, frac_mat, HKL, centric, bins):
    n_hkl = Emean.shape[0]
    n_atoms = xyz_ort.shape[0]
    npad = ((n_hkl + R - 1) // R) * R
    nt = npad // R

    frac = jnp.matmul(xyz_ort.astype(jnp.float32), frac_mat.astype(jnp.float32).T)
    frac8 = jnp.zeros((8, A_PAD), jnp.float32).at[:3, :n_atoms].set(frac.T)
    fvec = jnp.zeros((A_PAD, 8), jnp.float32).at[:n_atoms, 0].set(
        f_atoms.astype(jnp.float32))
    hkl8 = jnp.zeros((npad, 8), jnp.float32).at[:n_hkl, :3].set(
        HKL.astype(jnp.float32))

    def pad2d(v, fill):
        return jnp.full((npad,), fill, v.dtype).at[:n_hkl].set(v).reshape(npad // 128, 128)

    emean2d = pad2d(Emean.astype(jnp.float32), 0.5)
    dobs2d = pad2d(Dobs.astype(jnp.float32), 0.5)
    cent2d = pad2d(centric.astype(jnp.float32), 0.0)
    bins2d = pad2d(bins.astype(jnp.int32), N_BINS)

    f28 = pl.pallas_call(
        _f2_kernel,
        grid=(nt,),
        in_specs=[
            pl.BlockSpec((R, 8), lambda t: (t, 0)),
            pl.BlockSpec((8, A_PAD), lambda t: (0, 0)),
            pl.BlockSpec((A_PAD, 8), lambda t: (0, 0)),
        ],
        out_specs=pl.BlockSpec((R, 8), lambda t: (t, 0)),
        out_shape=jax.ShapeDtypeStruct((npad, 8), jnp.float32),
    )(hkl8, frac8, fvec)
    f2_2d = f28[:, 0].reshape(npad // 128, 128)

    out = pl.pallas_call(
        _stats_kernel,
        out_specs=pl.BlockSpec(memory_space=pltpu.SMEM),
        out_shape=jax.ShapeDtypeStruct((1, 1), jnp.float32),
    )(f2_2d, emean2d, dobs2d, cent2d, bins2d)
    return out[0, 0]


# speedup vs baseline: 1.3294x; 1.3294x over previous
import functools

import jax
import jax.numpy as jnp
import numpy as np
from jax.experimental import pallas as pl
from jax.experimental.pallas import tpu as pltpu

N_BINS = 10
A_PAD = 2048
R = 512
TWO_PI = np.float32(2.0 * np.pi)

_I0E_COEFFS_A = (
    -1.30002500998624804212E-8, 6.04699502254191894932E-8,
    -2.67079385394061173391E-7, 1.11738753912010371815E-6,
    -4.41673835845875056359E-6, 1.64484480707288970893E-5,
    -5.75419501008210370398E-5, 1.88502885095841655729E-4,
    -5.76375574538582365885E-4, 1.63947561694133579842E-3,
    -4.32430999505057594430E-3, 1.05464603945949983183E-2,
    -2.37374148058994688156E-2, 4.93052842396707084878E-2,
    -9.49010970480476444210E-2, 1.71620901522208775349E-1,
    -3.04682672343198398683E-1, 6.76795274409476084995E-1)
_I0E_COEFFS_B = (
    3.39623202570838634515E-9, 2.26666899049817806459E-8,
    2.04891858946906374183E-7, 2.89137052083475648297E-6,
    6.88975834691682398426E-5, 3.36911647825569408990E-3,
    8.04490411014108831608E-1)


def _chbevl(x, coefficients):
    b0 = jnp.zeros_like(x)
    b1 = jnp.zeros_like(x)
    b2 = jnp.zeros_like(x)
    for c in coefficients:
        b2 = b1
        b1 = b0
        b0 = x * b1 - b2 + jnp.float32(c)
    return 0.5 * (b0 - b2)


def _i0e(x):
    x = jnp.abs(x)
    result_le_8 = _chbevl(0.5 * x - 2.0, _I0E_COEFFS_A)
    result_gt_8 = _chbevl(32.0 / x - 2.0, _I0E_COEFFS_B) / jnp.sqrt(x)
    return jnp.where(x <= 8.0, result_le_8, result_gt_8)


def _log_i0(x):
    return jnp.log(_i0e(x)) + jnp.abs(x)


def _log_cosh(x):
    ax = jnp.abs(x)
    return ax + jnp.log1p(jnp.exp(-2.0 * ax)) - jnp.log(2.0).astype(jnp.float32)


def _f2_kernel(hkl_ref, frac_ref, fvec_ref, f2_ref):
    dn = (((1,), (0,)), ((), ()))
    praw = jax.lax.dot_general(hkl_ref[:, :], frac_ref[:, :], dn,
                               preferred_element_type=jnp.float32)
    phase = TWO_PI * praw
    c = jnp.cos(phase)
    s = jnp.sin(phase)
    fv = fvec_ref[:, :]
    f0 = fv.astype(jnp.bfloat16).astype(jnp.float32)
    r0 = fv - f0
    f1 = r0.astype(jnp.bfloat16).astype(jnp.float32)
    f2 = (r0 - f1).astype(jnp.bfloat16).astype(jnp.float32)

    def mv(m):
        d0 = jax.lax.dot_general(m, f0, dn, preferred_element_type=jnp.float32)
        d1 = jax.lax.dot_general(m, f1, dn, preferred_element_type=jnp.float32)
        d2 = jax.lax.dot_general(m, f2, dn, preferred_element_type=jnp.float32)
        return (d0 + d1) + d2

    A = mv(c)
    B = mv(s)
    f2_ref[:, :] = A * A + B * B


def _stats_kernel(f2_ref, emean_ref, dobs_ref, cent_ref, bins_ref, out_ref):
    if True:
        F2 = f2_ref[:, :]
        bins = bins_ref[:, :]
        Em = emean_ref[:, :]
        Do = dobs_ref[:, :]
        ce = cent_ref[:, :]
        Eobs2 = Em * Em

        zero = jnp.zeros_like(F2)
        ns, muF2s, muOs = [], [], []
        for b in range(N_BINS):
            oh = bins == b
            n = jnp.sum(jnp.where(oh, 1.0, 0.0))
            sF2 = jnp.sum(jnp.where(oh, F2, zero))
            sO = jnp.sum(jnp.where(oh, Eobs2, zero))
            ns.append(n)
            muF2s.append(sF2 / n)
            muOs.append(sO / n)
        g_sqF2 = jnp.ones_like(F2)
        g_muO = jnp.zeros_like(F2)
        for b in range(N_BINS):
            oh = bins == b
            g_sqF2 = jnp.where(oh, jnp.sqrt(muF2s[b]), g_sqF2)
            g_muO = jnp.where(oh, muOs[b], g_muO)
        Ecalc = jnp.sqrt(F2 + 1e-12) / g_sqF2
        Ecalc2 = Ecalc * Ecalc
        Eoi = Eobs2 - g_muO

        muCs = []
        for b in range(N_BINS):
            oh = bins == b
            muCs.append(jnp.sum(jnp.where(oh, Ecalc2, zero)) / ns[b])
        g_muC = jnp.zeros_like(F2)
        for b in range(N_BINS):
            g_muC = jnp.where(bins == b, muCs[b], g_muC)
        Eci = Ecalc2 - g_muC

        g_sig = jnp.full_like(F2, 0.5)
        for b in range(N_BINS):
            oh = bins == b
            n = ns[b]
            cov = jnp.sum(jnp.where(oh, Eci * Eoi, zero)) / (n - 1.0)
            m_o = jnp.sum(jnp.where(oh, Eoi, zero)) / n
            m_c = jnp.sum(jnp.where(oh, Eci, zero)) / n
            sso = jnp.sum(jnp.where(oh, Eoi * Eoi, zero)) / n
            ssc = jnp.sum(jnp.where(oh, Eci * Eci, zero)) / n
            std_o = jnp.sqrt(sso - m_o * m_o)
            std_c = jnp.sqrt(ssc - m_c * m_c)
            sig = jnp.sqrt(jnp.clip(cov / (std_o * std_c), 0.001, 0.999))
            g_sig = jnp.where(oh, sig, g_sig)

        dsA = Do * g_sig
        t2 = dsA * dsA
        den = 1.0 - t2
        e2sum = Em ** 2 + Ecalc ** 2
        barg = 2.0 * Em * Ecalc * dsA / den
        llg_a = -jnp.log(den) - t2 * e2sum / den + _log_i0(barg)
        carg = Em * Ecalc * dsA / den
        llg_c = -0.5 * jnp.log(den) - t2 * e2sum / (2.0 * den) + _log_cosh(carg)
        llg = jnp.where(ce > 0.5, llg_c, llg_a)
        total = jnp.sum(jnp.where(bins < N_BINS, llg, zero))
        out_ref[0, 0] = total


def kernel(xyz_ort, f_atoms, Emean, Dobs, frac_mat, HKL, centric, bins):
    n_hkl = Emean.shape[0]
    n_atoms = xyz_ort.shape[0]
    npad = ((n_hkl + R - 1) // R) * R
    nt = npad // R

    frac = jnp.matmul(xyz_ort.astype(jnp.float32), frac_mat.astype(jnp.float32).T)
    frac8 = jnp.zeros((8, A_PAD), jnp.float32).at[:3, :n_atoms].set(frac.T)
    fvec = jnp.zeros((A_PAD, 8), jnp.float32).at[:n_atoms, 0].set(
        f_atoms.astype(jnp.float32))
    hkl8 = jnp.zeros((npad, 8), jnp.float32).at[:n_hkl, :3].set(
        HKL.astype(jnp.float32))

    def pad2d(v, fill):
        return jnp.full((npad,), fill, v.dtype).at[:n_hkl].set(v).reshape(npad // 128, 128)

    emean2d = pad2d(Emean.astype(jnp.float32), 0.5)
    dobs2d = pad2d(Dobs.astype(jnp.float32), 0.5)
    cent2d = pad2d(centric.astype(jnp.float32), 0.0)
    bins2d = pad2d(bins.astype(jnp.int32), N_BINS)

    f28 = pl.pallas_call(
        _f2_kernel,
        grid=(nt,),
        in_specs=[
            pl.BlockSpec((R, 8), lambda t: (t, 0)),
            pl.BlockSpec((8, A_PAD), lambda t: (0, 0)),
            pl.BlockSpec((A_PAD, 8), lambda t: (0, 0)),
        ],
        out_specs=pl.BlockSpec((R, 8), lambda t: (t, 0)),
        out_shape=jax.ShapeDtypeStruct((npad, 8), jnp.float32),
    )(hkl8, frac8, fvec)
    f2_2d = f28[:, 0].reshape(npad // 128, 128)

    out = pl.pallas_call(
        _stats_kernel,
        out_specs=pl.BlockSpec(memory_space=pltpu.SMEM),
        out_shape=jax.ShapeDtypeStruct((1, 1), jnp.float32),
    )(f2_2d, emean2d, dobs2d, cent2d, bins2d)
    return out[0, 0]


# speedup vs baseline: 1.3414x; 1.0090x over previous
import functools

import jax
import jax.numpy as jnp
import numpy as np
from jax.experimental import pallas as pl
from jax.experimental.pallas import tpu as pltpu

N_BINS = 10
A_PAD = 2048
R = 640
TWO_PI = np.float32(2.0 * np.pi)

_I0E_COEFFS_A = (
    -1.30002500998624804212E-8, 6.04699502254191894932E-8,
    -2.67079385394061173391E-7, 1.11738753912010371815E-6,
    -4.41673835845875056359E-6, 1.64484480707288970893E-5,
    -5.75419501008210370398E-5, 1.88502885095841655729E-4,
    -5.76375574538582365885E-4, 1.63947561694133579842E-3,
    -4.32430999505057594430E-3, 1.05464603945949983183E-2,
    -2.37374148058994688156E-2, 4.93052842396707084878E-2,
    -9.49010970480476444210E-2, 1.71620901522208775349E-1,
    -3.04682672343198398683E-1, 6.76795274409476084995E-1)
_I0E_COEFFS_B = (
    3.39623202570838634515E-9, 2.26666899049817806459E-8,
    2.04891858946906374183E-7, 2.89137052083475648297E-6,
    6.88975834691682398426E-5, 3.36911647825569408990E-3,
    8.04490411014108831608E-1)


def _chbevl(x, coefficients):
    b0 = jnp.zeros_like(x)
    b1 = jnp.zeros_like(x)
    b2 = jnp.zeros_like(x)
    for c in coefficients:
        b2 = b1
        b1 = b0
        b0 = x * b1 - b2 + jnp.float32(c)
    return 0.5 * (b0 - b2)


def _i0e(x):
    x = jnp.abs(x)
    result_le_8 = _chbevl(0.5 * x - 2.0, _I0E_COEFFS_A)
    result_gt_8 = _chbevl(32.0 / x - 2.0, _I0E_COEFFS_B) / jnp.sqrt(x)
    return jnp.where(x <= 8.0, result_le_8, result_gt_8)


def _log_i0(x):
    return jnp.log(_i0e(x)) + jnp.abs(x)


def _log_cosh(x):
    ax = jnp.abs(x)
    return ax + jnp.log1p(jnp.exp(-2.0 * ax)) - jnp.log(2.0).astype(jnp.float32)


def _f2_kernel(hkl_ref, frac_ref, fvec_ref, f2_ref):
    dn = (((1,), (0,)), ((), ()))
    praw = jax.lax.dot_general(hkl_ref[:, :], frac_ref[:, :], dn,
                               preferred_element_type=jnp.float32)
    phase = TWO_PI * praw
    c = jnp.cos(phase)
    s = jnp.sin(phase)
    fv = fvec_ref[:, :]
    f0 = fv.astype(jnp.bfloat16).astype(jnp.float32)
    r0 = fv - f0
    f1 = r0.astype(jnp.bfloat16).astype(jnp.float32)
    f2 = (r0 - f1).astype(jnp.bfloat16).astype(jnp.float32)

    def mv(m):
        d0 = jax.lax.dot_general(m, f0, dn, preferred_element_type=jnp.float32)
        d1 = jax.lax.dot_general(m, f1, dn, preferred_element_type=jnp.float32)
        d2 = jax.lax.dot_general(m, f2, dn, preferred_element_type=jnp.float32)
        return (d0 + d1) + d2

    A = mv(c)
    B = mv(s)
    f2_ref[:, :] = A * A + B * B


def _stats_kernel(f2_ref, emean_ref, dobs_ref, cent_ref, bins_ref, out_ref):
    if True:
        F2 = f2_ref[:, :]
        bins = bins_ref[:, :]
        Em = emean_ref[:, :]
        Do = dobs_ref[:, :]
        ce = cent_ref[:, :]
        Eobs2 = Em * Em

        zero = jnp.zeros_like(F2)
        ns, muF2s, muOs = [], [], []
        for b in range(N_BINS):
            oh = bins == b
            n = jnp.sum(jnp.where(oh, 1.0, 0.0))
            sF2 = jnp.sum(jnp.where(oh, F2, zero))
            sO = jnp.sum(jnp.where(oh, Eobs2, zero))
            ns.append(n)
            muF2s.append(sF2 / n)
            muOs.append(sO / n)
        g_sqF2 = jnp.ones_like(F2)
        g_muO = jnp.zeros_like(F2)
        for b in range(N_BINS):
            oh = bins == b
            g_sqF2 = jnp.where(oh, jnp.sqrt(muF2s[b]), g_sqF2)
            g_muO = jnp.where(oh, muOs[b], g_muO)
        Ecalc = jnp.sqrt(F2 + 1e-12) / g_sqF2
        Ecalc2 = Ecalc * Ecalc
        Eoi = Eobs2 - g_muO

        muCs = []
        for b in range(N_BINS):
            oh = bins == b
            muCs.append(jnp.sum(jnp.where(oh, Ecalc2, zero)) / ns[b])
        g_muC = jnp.zeros_like(F2)
        for b in range(N_BINS):
            g_muC = jnp.where(bins == b, muCs[b], g_muC)
        Eci = Ecalc2 - g_muC

        g_sig = jnp.full_like(F2, 0.5)
        for b in range(N_BINS):
            oh = bins == b
            n = ns[b]
            cov = jnp.sum(jnp.where(oh, Eci * Eoi, zero)) / (n - 1.0)
            m_o = jnp.sum(jnp.where(oh, Eoi, zero)) / n
            m_c = jnp.sum(jnp.where(oh, Eci, zero)) / n
            sso = jnp.sum(jnp.where(oh, Eoi * Eoi, zero)) / n
            ssc = jnp.sum(jnp.where(oh, Eci * Eci, zero)) / n
            std_o = jnp.sqrt(sso - m_o * m_o)
            std_c = jnp.sqrt(ssc - m_c * m_c)
            sig = jnp.sqrt(jnp.clip(cov / (std_o * std_c), 0.001, 0.999))
            g_sig = jnp.where(oh, sig, g_sig)

        dsA = Do * g_sig
        t2 = dsA * dsA
        den = 1.0 - t2
        e2sum = Em ** 2 + Ecalc ** 2
        barg = 2.0 * Em * Ecalc * dsA / den
        llg_a = -jnp.log(den) - t2 * e2sum / den + _log_i0(barg)
        carg = Em * Ecalc * dsA / den
        llg_c = -0.5 * jnp.log(den) - t2 * e2sum / (2.0 * den) + _log_cosh(carg)
        llg = jnp.where(ce > 0.5, llg_c, llg_a)
        total = jnp.sum(jnp.where(bins < N_BINS, llg, zero))
        out_ref[0, 0] = total


def kernel(xyz_ort, f_atoms, Emean, Dobs, frac_mat, HKL, centric, bins):
    n_hkl = Emean.shape[0]
    n_atoms = xyz_ort.shape[0]
    npad = ((n_hkl + R - 1) // R) * R
    nt = npad // R

    frac = jnp.matmul(xyz_ort.astype(jnp.float32), frac_mat.astype(jnp.float32).T)
    frac8 = jnp.zeros((8, A_PAD), jnp.float32).at[:3, :n_atoms].set(frac.T)
    fvec = jnp.zeros((A_PAD, 8), jnp.float32).at[:n_atoms, 0].set(
        f_atoms.astype(jnp.float32))
    hkl8 = jnp.zeros((npad, 8), jnp.float32).at[:n_hkl, :3].set(
        HKL.astype(jnp.float32))

    def pad2d(v, fill):
        return jnp.full((npad,), fill, v.dtype).at[:n_hkl].set(v).reshape(npad // 128, 128)

    emean2d = pad2d(Emean.astype(jnp.float32), 0.5)
    dobs2d = pad2d(Dobs.astype(jnp.float32), 0.5)
    cent2d = pad2d(centric.astype(jnp.float32), 0.0)
    bins2d = pad2d(bins.astype(jnp.int32), N_BINS)

    f28 = pl.pallas_call(
        _f2_kernel,
        grid=(nt,),
        in_specs=[
            pl.BlockSpec((R, 8), lambda t: (t, 0)),
            pl.BlockSpec((8, A_PAD), lambda t: (0, 0)),
            pl.BlockSpec((A_PAD, 8), lambda t: (0, 0)),
        ],
        out_specs=pl.BlockSpec((R, 8), lambda t: (t, 0)),
        out_shape=jax.ShapeDtypeStruct((npad, 8), jnp.float32),
    )(hkl8, frac8, fvec)
    f2_2d = f28[:, 0].reshape(npad // 128, 128)

    out = pl.pallas_call(
        _stats_kernel,
        out_specs=pl.BlockSpec(memory_space=pltpu.SMEM),
        out_shape=jax.ShapeDtypeStruct((1, 1), jnp.float32),
    )(f2_2d, emean2d, dobs2d, cent2d, bins2d)
    return out[0, 0]


# speedup vs baseline: 1.3500x; 1.0064x over previous
import jax
import jax.numpy as jnp
import numpy as np
from jax.experimental import pallas as pl
from jax.experimental.pallas import tpu as pltpu

N_BINS = 10
A_PAD = 2048
R = 512
TWO_PI = np.float32(2.0 * np.pi)

_I0E_COEFFS_A = (
    -1.30002500998624804212E-8, 6.04699502254191894932E-8,
    -2.67079385394061173391E-7, 1.11738753912010371815E-6,
    -4.41673835845875056359E-6, 1.64484480707288970893E-5,
    -5.75419501008210370398E-5, 1.88502885095841655729E-4,
    -5.76375574538582365885E-4, 1.63947561694133579842E-3,
    -4.32430999505057594430E-3, 1.05464603945949983183E-2,
    -2.37374148058994688156E-2, 4.93052842396707084878E-2,
    -9.49010970480476444210E-2, 1.71620901522208775349E-1,
    -3.04682672343198398683E-1, 6.76795274409476084995E-1)
_I0E_COEFFS_B = (
    3.39623202570838634515E-9, 2.26666899049817806459E-8,
    2.04891858946906374183E-7, 2.89137052083475648297E-6,
    6.88975834691682398426E-5, 3.36911647825569408990E-3,
    8.04490411014108831608E-1)


def _chbevl(x, coefficients):
    b0 = jnp.zeros_like(x)
    b1 = jnp.zeros_like(x)
    b2 = jnp.zeros_like(x)
    for c in coefficients:
        b2 = b1
        b1 = b0
        b0 = x * b1 - b2 + jnp.float32(c)
    return 0.5 * (b0 - b2)


def _i0e(x):
    x = jnp.abs(x)
    result_le_8 = _chbevl(0.5 * x - 2.0, _I0E_COEFFS_A)
    result_gt_8 = _chbevl(32.0 / x - 2.0, _I0E_COEFFS_B) / jnp.sqrt(x)
    return jnp.where(x <= 8.0, result_le_8, result_gt_8)


def _log_i0(x):
    return jnp.log(_i0e(x)) + jnp.abs(x)


def _log_cosh(x):
    ax = jnp.abs(x)
    return ax + jnp.log1p(jnp.exp(-2.0 * ax)) - jnp.log(2.0).astype(jnp.float32)


def _cs_kernel(hkl_ref, frac_ref, c_ref, s_ref):
    dn = (((1,), (0,)), ((), ()))
    praw = jax.lax.dot_general(hkl_ref[:, :], frac_ref[:, :], dn,
                               preferred_element_type=jnp.float32)
    phase = TWO_PI * praw
    c_ref[:, :] = jnp.cos(phase)
    s_ref[:, :] = jnp.sin(phase)


def _stats_kernel(f2_ref, emean_ref, dobs_ref, cent_ref, bins_ref, out_ref):
    F2 = f2_ref[:, :]
    bins = bins_ref[:, :]
    Em = emean_ref[:, :]
    Do = dobs_ref[:, :]
    ce = cent_ref[:, :]
    Eobs2 = Em * Em

    zero = jnp.zeros_like(F2)
    ns, muF2s, muOs = [], [], []
    for b in range(N_BINS):
        oh = bins == b
        n = jnp.sum(jnp.where(oh, 1.0, 0.0))
        sF2 = jnp.sum(jnp.where(oh, F2, zero))
        sO = jnp.sum(jnp.where(oh, Eobs2, zero))
        ns.append(n)
        muF2s.append(sF2 / n)
        muOs.append(sO / n)
    g_sqF2 = jnp.ones_like(F2)
    g_muO = jnp.zeros_like(F2)
    for b in range(N_BINS):
        oh = bins == b
        g_sqF2 = jnp.where(oh, jnp.sqrt(muF2s[b]), g_sqF2)
        g_muO = jnp.where(oh, muOs[b], g_muO)
    Ecalc = jnp.sqrt(F2 + 1e-12) / g_sqF2
    Ecalc2 = Ecalc * Ecalc
    Eoi = Eobs2 - g_muO

    muCs = []
    for b in range(N_BINS):
        oh = bins == b
        muCs.append(jnp.sum(jnp.where(oh, Ecalc2, zero)) / ns[b])
    g_muC = jnp.zeros_like(F2)
    for b in range(N_BINS):
        g_muC = jnp.where(bins == b, muCs[b], g_muC)
    Eci = Ecalc2 - g_muC

    g_sig = jnp.full_like(F2, 0.5)
    for b in range(N_BINS):
        oh = bins == b
        n = ns[b]
        cov = jnp.sum(jnp.where(oh, Eci * Eoi, zero)) / (n - 1.0)
        m_o = jnp.sum(jnp.where(oh, Eoi, zero)) / n
        m_c = jnp.sum(jnp.where(oh, Eci, zero)) / n
        sso = jnp.sum(jnp.where(oh, Eoi * Eoi, zero)) / n
        ssc = jnp.sum(jnp.where(oh, Eci * Eci, zero)) / n
        std_o = jnp.sqrt(sso - m_o * m_o)
        std_c = jnp.sqrt(ssc - m_c * m_c)
        sig = jnp.sqrt(jnp.clip(cov / (std_o * std_c), 0.001, 0.999))
        g_sig = jnp.where(oh, sig, g_sig)

    dsA = Do * g_sig
    t2 = dsA * dsA
    den = 1.0 - t2
    e2sum = Em ** 2 + Ecalc ** 2
    barg = 2.0 * Em * Ecalc * dsA / den
    llg_a = -jnp.log(den) - t2 * e2sum / den + _log_i0(barg)
    carg = Em * Ecalc * dsA / den
    llg_c = -0.5 * jnp.log(den) - t2 * e2sum / (2.0 * den) + _log_cosh(carg)
    llg = jnp.where(ce > 0.5, llg_c, llg_a)
    total = jnp.sum(jnp.where(bins < N_BINS, llg, zero))
    out_ref[0, 0] = total


def kernel(xyz_ort, f_atoms, Emean, Dobs, frac_mat, HKL, centric, bins):
    n_hkl = Emean.shape[0]
    n_atoms = xyz_ort.shape[0]
    npad = ((n_hkl + R - 1) // R) * R
    nt = npad // R

    frac = jnp.matmul(xyz_ort.astype(jnp.float32), frac_mat.astype(jnp.float32).T)
    frac8 = jnp.zeros((8, A_PAD), jnp.float32).at[:3, :n_atoms].set(frac.T)
    hkl8 = jnp.zeros((npad, 8), jnp.float32).at[:n_hkl, :3].set(
        HKL.astype(jnp.float32))

    c, s = pl.pallas_call(
        _cs_kernel,
        grid=(nt,),
        in_specs=[
            pl.BlockSpec((R, 8), lambda t: (t, 0)),
            pl.BlockSpec((8, A_PAD), lambda t: (0, 0)),
        ],
        out_specs=[
            pl.BlockSpec((R, A_PAD), lambda t: (t, 0)),
            pl.BlockSpec((R, A_PAD), lambda t: (t, 0)),
        ],
        out_shape=[
            jax.ShapeDtypeStruct((npad, A_PAD), jnp.float32),
            jax.ShapeDtypeStruct((npad, A_PAD), jnp.float32),
        ],
    )(hkl8, frac8)

    f32f = f_atoms.astype(jnp.float32)
    A = c[:n_hkl, :n_atoms] @ f32f
    B = s[:n_hkl, :n_atoms] @ f32f
    F2 = A * A + B * B

    def pad2d(v, fill):
        return jnp.full((npad,), fill, v.dtype).at[:n_hkl].set(v).reshape(
            npad // 128, 128)

    f2_2d = pad2d(F2, 0.0)
    emean2d = pad2d(Emean.astype(jnp.float32), 0.5)
    dobs2d = pad2d(Dobs.astype(jnp.float32), 0.5)
    cent2d = pad2d(centric.astype(jnp.float32), 0.0)
    bins2d = pad2d(bins.astype(jnp.int32), N_BINS)

    out = pl.pallas_call(
        _stats_kernel,
        out_specs=pl.BlockSpec(memory_space=pltpu.SMEM),
        out_shape=jax.ShapeDtypeStruct((1, 1), jnp.float32),
    )(f2_2d, emean2d, dobs2d, cent2d, bins2d)
    return out[0, 0]
